# Initial kernel scaffold; baseline (speedup 1.0000x reference)
import jax
import jax.numpy as jnp
from jax.experimental import pallas as pl


def kernel(x, path_indices_0, path_indices_1, path_indices_2):
    raise NotImplementedError("write your pallas kernel here")


# speedup vs baseline: 7.4282x; 7.4282x over previous
import functools

import jax
import jax.numpy as jnp
from jax import lax
from jax.experimental import pallas as pl
from jax.experimental.pallas import tpu as pltpu
from jax.experimental.pallas import tpu_sc as plsc

B, D, H, W = 16, 8, 56, 56
M = D * H * W
WA = H * W
PATHS = ((4, 2), (8, 3), (12, 4))
NW = 32
CH = 392
LMAX = 4

_SCHEDULE = []
_idx_rows = 0
_type_row_base = 0
for _t, (_P, _L) in enumerate(PATHS):
    _C = _P * WA // NW
    for _j in range(_C // CH):
        _SCHEDULE.append((_t, _j, _L, _idx_rows, _type_row_base, _C))
        _idx_rows += _L
    _type_row_base += _P * WA
N_IDX_ROWS = _idx_rows
N_CHUNKS = len(_SCHEDULE)
NOUT = sum(_P * WA for (_P, _L) in PATHS)


def _sc_body(edge_hbm, idx0_hbm, idx1_hbm, idx2_hbm, out_hbm,
             idx_v, rows_v, out_v, sem_i, sem_g0, sem_g1, sem_s):
    idx_hbms = (idx0_hbm, idx1_hbm, idx2_hbm)
    sem_g = (sem_g0, sem_g1)
    wid = lax.axis_index("s") * 2 + lax.axis_index("c")

    idx_handles = []
    for (t, j, L, irow, trb, C) in _SCHEDULE:
        nc = C // CH
        g = wid * nc + j
        p = g // 8
        w0 = (g % 8) * CH
        for l in range(L):
            off = (p * L + l) * WA + w0
            idx_handles.append(pltpu.async_copy(
                idx_hbms[t].at[pl.ds(off, CH)],
                idx_v.at[irow + l, pl.ds(0, CH)], sem_i))
    for h in idx_handles:
        h.wait()

    def fire_gathers(k):
        t, j, L, irow, trb, C = _SCHEDULE[k]
        pk = k % 2
        hs = []
        for l in range(L):
            hs.append(pltpu.async_copy(
                edge_hbm.at[idx_v.at[irow + l, pl.ds(0, CH)]],
                rows_v.at[pk, l], sem_g[pk]))
        return hs

    def compute(k):
        t, j, L, irow, trb, C = _SCHEDULE[k]
        pk = k % 2

        def body(i, carry):
            v = rows_v[pk, 0, i, :]
            for l in range(1, L):
                v = jnp.maximum(v, rows_v[pk, l, i, :])
            out_v[pk, i, :] = 1.0 - v
            return carry

        lax.fori_loop(0, CH, body, 0, unroll=4)

    def fire_scatter(k):
        t, j, L, irow, trb, C = _SCHEDULE[k]
        pk = k % 2
        row = trb + wid * C + j * CH
        return pltpu.async_copy(
            out_v.at[pk], out_hbm.at[pl.ds(row, CH)], sem_s)

    gather_h = {0: fire_gathers(0)}
    scatter_h = {}
    for k in range(N_CHUNKS):
        if k + 1 < N_CHUNKS:
            gather_h[k + 1] = fire_gathers(k + 1)
        for h in gather_h.pop(k):
            h.wait()
        if k - 2 in scatter_h:
            scatter_h.pop(k - 2).wait()
        compute(k)
        scatter_h[k] = fire_scatter(k)
    for h in scatter_h.values():
        h.wait()


@functools.partial(jax.jit, static_argnums=())
def _sc_call(edge_t, i0, i1, i2):
    mesh = plsc.VectorSubcoreMesh(core_axis_name="c", subcore_axis_name="s")
    return pl.kernel(
        _sc_body,
        out_type=jax.ShapeDtypeStruct((NOUT, B), jnp.float32),
        mesh=mesh,
        scratch_types=[
            pltpu.VMEM((N_IDX_ROWS, 512), jnp.int32),
            pltpu.VMEM((2, LMAX, CH, B), jnp.float32),
            pltpu.VMEM((2, CH, B), jnp.float32),
            pltpu.SemaphoreType.DMA,
            pltpu.SemaphoreType.DMA,
            pltpu.SemaphoreType.DMA,
            pltpu.SemaphoreType.DMA,
        ],
        compiler_params=pltpu.CompilerParams(use_tc_tiling_on_sc=False),
    )(edge_t, i0, i1, i2)


def kernel(x, path_indices_0, path_indices_1, path_indices_2):
    edge_t = x.reshape(B, M).T
    out_t = _sc_call(edge_t,
                     path_indices_0.reshape(-1),
                     path_indices_1.reshape(-1),
                     path_indices_2.reshape(-1))
    return out_t.T.reshape(B, sum(p for p, _ in PATHS), WA)


# speedup vs baseline: 11.2548x; 1.5151x over previous
import functools

import jax
import jax.numpy as jnp
from jax import lax
from jax.experimental import pallas as pl
from jax.experimental.pallas import tpu as pltpu
from jax.experimental.pallas import tpu_sc as plsc

B, D, H, W = 16, 8, 56, 56
M = D * H * W
WA = H * W
HW = WA // 2
NVEC = HW // 16
PATHS = ((4, 2), (8, 3), (12, 4))
NPG = sum(p for p, _ in PATHS)
NOUT = NPG * WA
LMAX = 4

_PATH_OF = []
for _t, (_P, _L) in enumerate(PATHS):
    for _p in range(_P):
        _PATH_OF.append((_t, _p, _L))


def _sc_body(x_hbm, i0_hbm, i1_hbm, i2_hbm, out_hbm,
             tab_v, idx_v, out_v, sem_t, sem_i0, sem_i1, sem_o):
    idx_hbms = (i0_hbm, i1_hbm, i2_hbm)
    sem_i = (sem_i0, sem_i1)
    b = lax.axis_index("s")
    h = lax.axis_index("c")
    w0 = h * HW

    tab_h = pltpu.async_copy(x_hbm.at[pl.ds(b * M, M)], tab_v, sem_t)

    def fire_idx(pg):
        t, p, L = _PATH_OF[pg]
        pk = pg % 2
        hs = []
        for l in range(L):
            off = (p * L + l) * WA + w0
            hs.append(pltpu.async_copy(
                idx_hbms[t].at[pl.ds(off, HW)], idx_v.at[pk, l], sem_i[pk]))
        return hs

    def compute(pg):
        t, p, L = _PATH_OF[pg]
        pk = pg % 2

        def body(g, carry):
            s = pl.ds(g * 16, 16)
            v = plsc.load_gather(tab_v, [idx_v[pk, 0, s]])
            for l in range(1, L):
                v = jnp.maximum(v, plsc.load_gather(tab_v, [idx_v[pk, l, s]]))
            out_v[pk, s] = 1.0 - v
            return carry

        lax.fori_loop(0, NVEC, body, 0, unroll=4)

    def fire_out(pg):
        pk = pg % 2
        col = b * NOUT + pg * WA + w0
        return pltpu.async_copy(out_v.at[pk], out_hbm.at[pl.ds(col, HW)],
                                sem_o)

    idx_h = {0: fire_idx(0)}
    out_h = {}
    tab_waited = False
    for pg in range(NPG):
        if pg + 1 < NPG:
            idx_h[pg + 1] = fire_idx(pg + 1)
        for hnd in idx_h.pop(pg):
            hnd.wait()
        if not tab_waited:
            tab_h.wait()
            tab_waited = True
        if pg - 2 in out_h:
            out_h.pop(pg - 2).wait()
        compute(pg)
        out_h[pg] = fire_out(pg)
    for hnd in out_h.values():
        hnd.wait()


@jax.jit
def _sc_call(x_flat, i0, i1, i2):
    mesh = plsc.VectorSubcoreMesh(core_axis_name="c", subcore_axis_name="s")
    return pl.kernel(
        _sc_body,
        out_type=jax.ShapeDtypeStruct((B * NOUT,), jnp.float32),
        mesh=mesh,
        scratch_types=[
            pltpu.VMEM((M,), jnp.float32),
            pltpu.VMEM((2, LMAX, HW), jnp.int32),
            pltpu.VMEM((2, HW), jnp.float32),
            pltpu.SemaphoreType.DMA,
            pltpu.SemaphoreType.DMA,
            pltpu.SemaphoreType.DMA,
            pltpu.SemaphoreType.DMA,
        ],
        compiler_params=pltpu.CompilerParams(
            use_tc_tiling_on_sc=False, needs_layout_passes=False),
    )(x_flat, i0, i1, i2)


def kernel(x, path_indices_0, path_indices_1, path_indices_2):
    out = _sc_call(x.reshape(-1),
                   path_indices_0.reshape(-1),
                   path_indices_1.reshape(-1),
                   path_indices_2.reshape(-1))
    return out.reshape(B, NPG, WA)


# speedup vs baseline: 12.9089x; 1.1470x over previous
import functools

import jax
import jax.numpy as jnp
from jax import lax
from jax.experimental import pallas as pl
from jax.experimental.pallas import tpu as pltpu
from jax.experimental.pallas import tpu_sc as plsc

B, D, H, W = 16, 8, 56, 56
M = D * H * W
WA = H * W
HW = WA // 2
NVEC = HW // 16
PATHS = ((4, 2), (8, 3), (12, 4))
NPG = sum(p for p, _ in PATHS)
NOUT = NPG * WA
LMAX = 4

_PATH_OF = []
for _t, (_P, _L) in enumerate(PATHS):
    for _p in range(_P):
        _PATH_OF.append((_t, _p, _L))


def _sc_body(x_hbm, i0_hbm, i1_hbm, i2_hbm, out_hbm,
             tab_v, idx_v, out_v, sem_t, sem_i0, sem_i1, sem_o):
    idx_hbms = (i0_hbm, i1_hbm, i2_hbm)
    sem_i = (sem_i0, sem_i1)
    b = lax.axis_index("s")
    h = lax.axis_index("c")
    w0 = h * HW

    tab_h = pltpu.async_copy(x_hbm.at[pl.ds(b * M, M)], tab_v, sem_t)

    def fire_idx(pg):
        t, p, L = _PATH_OF[pg]
        pk = pg % 2
        hs = []
        for l in range(L):
            off = (p * L + l) * WA + w0
            hs.append(pltpu.async_copy(
                idx_hbms[t].at[pl.ds(off, HW)], idx_v.at[pk, l], sem_i[pk]))
        return hs

    def compute(pg):
        t, p, L = _PATH_OF[pg]
        pk = pg % 2

        @plsc.parallel_loop(0, HW, 16, unroll=7)
        def body(g):
            s = pl.ds(g, 16)
            v = plsc.load_gather(tab_v, [idx_v[pk, 0, s]])
            for l in range(1, L):
                v = jnp.maximum(v, plsc.load_gather(tab_v, [idx_v[pk, l, s]]))
            out_v[pk, s] = 1.0 - v

    def fire_out(pg):
        pk = pg % 2
        col = b * NOUT + pg * WA + w0
        return pltpu.async_copy(out_v.at[pk], out_hbm.at[pl.ds(col, HW)],
                                sem_o)

    idx_h = {0: fire_idx(0)}
    out_h = {}
    tab_waited = False
    for pg in range(NPG):
        if pg + 1 < NPG:
            idx_h[pg + 1] = fire_idx(pg + 1)
        for hnd in idx_h.pop(pg):
            hnd.wait()
        if not tab_waited:
            tab_h.wait()
            tab_waited = True
        if pg - 2 in out_h:
            out_h.pop(pg - 2).wait()
        compute(pg)
        out_h[pg] = fire_out(pg)
    for hnd in out_h.values():
        hnd.wait()


@jax.jit
def _sc_call(x_flat, i0, i1, i2):
    mesh = plsc.VectorSubcoreMesh(core_axis_name="c", subcore_axis_name="s")
    return pl.kernel(
        _sc_body,
        out_type=jax.ShapeDtypeStruct((B * NOUT,), jnp.float32),
        mesh=mesh,
        scratch_types=[
            pltpu.VMEM((M,), jnp.float32),
            pltpu.VMEM((2, LMAX, HW), jnp.int32),
            pltpu.VMEM((2, HW), jnp.float32),
            pltpu.SemaphoreType.DMA,
            pltpu.SemaphoreType.DMA,
            pltpu.SemaphoreType.DMA,
            pltpu.SemaphoreType.DMA,
        ],
        compiler_params=pltpu.CompilerParams(
            use_tc_tiling_on_sc=False, needs_layout_passes=False),
    )(x_flat, i0, i1, i2)


def kernel(x, path_indices_0, path_indices_1, path_indices_2):
    out = _sc_call(x.reshape(-1),
                   path_indices_0.reshape(-1),
                   path_indices_1.reshape(-1),
                   path_indices_2.reshape(-1))
    return out.reshape(B, NPG, WA)
